# Initial kernel scaffold; baseline (speedup 1.0000x reference)
import jax
import jax.numpy as jnp
from jax.experimental import pallas as pl


def kernel(inputs, inputs2, indexes, source_classes, features, labels):
    raise NotImplementedError("write your pallas kernel here")


# speedup vs baseline: 2.5590x; 2.5590x over previous
import functools

import jax
import jax.numpy as jnp
from jax.experimental import pallas as pl
from jax.experimental.pallas import tpu as pltpu

TEMP = 0.05
B, D, M, C = 512, 128, 65536, 1000
TM = 1024
NB = M // TM
S = 500
NEG = -3e38
POS = 3e38


def _body(feat_ref, lab_s_ref, lab_v_ref, in1_ref, in2_ref, idx_ref,
          o_con_ref, o_mse_ref,
          amax, amin, afsum, anum, lg_s, tgt):
    i = pl.program_id(0)

    @pl.when(i == 0)
    def _init():
        amax[...] = jnp.full_like(amax, NEG)
        amin[...] = jnp.full_like(amin, POS)
        afsum[...] = jnp.zeros_like(afsum)
        anum[...] = jnp.zeros_like(anum)
        tgt[...] = jnp.zeros_like(tgt)

    lg_s[...] = jax.lax.dot_general(
        feat_ref[...], in1_ref[...],
        dimension_numbers=(((1,), (1,)), ((), ())),
        preferred_element_type=jnp.float32) * jnp.float32(1.0 / TEMP)

    lane = jax.lax.broadcasted_iota(jnp.int32, (B, TM), 1) + i * TM
    oh = (idx_ref[...] == lane).astype(jnp.float32)
    tgt[...] += jax.lax.dot_general(
        lab_v_ref[0], oh,
        dimension_numbers=(((1,), (1,)), ((), ())),
        preferred_element_type=jnp.float32)

    def mstep(m, _):
        c = lab_s_ref[0, 0, m].astype(jnp.int32)
        row = lg_s[pl.ds(m, 1), :]
        amax[pl.ds(c, 1), :] = jnp.maximum(amax[pl.ds(c, 1), :], row)
        amin[pl.ds(c, 1), :] = jnp.minimum(amin[pl.ds(c, 1), :], row)
        afsum[pl.ds(c, 1), :] = afsum[pl.ds(c, 1), :] + feat_ref[pl.ds(m, 1), :]
        anum[pl.ds(c, 1), :] = anum[pl.ds(c, 1), :] + 1.0
        return 0
    jax.lax.fori_loop(0, TM, mstep, 0)

    @pl.when(i == NB - 1)
    def _epilogue():
        nums = anum[...]
        inv_tn = jnp.float32(1.0 / TEMP) / nums
        simT = jax.lax.dot_general(
            afsum[...], in1_ref[...],
            dimension_numbers=(((1,), (1,)), ((), ())),
            preferred_element_type=jnp.float32) * inv_tn
        sim2T = jax.lax.dot_general(
            afsum[...], in2_ref[...],
            dimension_numbers=(((1,), (1,)), ((), ())),
            preferred_element_type=jnp.float32) * inv_tn

        def colsoftmax(p):
            mx = jnp.max(p, axis=0, keepdims=True)
            e = jnp.exp(p - mx)
            return e / jnp.sum(e, axis=0, keepdims=True)

        dsm = colsoftmax(simT[:S, :]) - colsoftmax(sim2T[:S, :])
        o_mse_ref[...] = jnp.sum(dsm * dsm).reshape(1, 1)

        tgt_i = tgt[...].astype(jnp.int32)
        cio = jax.lax.broadcasted_iota(jnp.int32, (C, B), 0)
        ohc = (cio == tgt_i).astype(jnp.float32)
        av = amax[...]
        nv = amin[...]
        tmax = jnp.sum(av * ohc, axis=0, keepdims=True)
        tmin = jnp.sum(nv * ohc, axis=0, keepdims=True)
        sumexp = jnp.sum(jnp.exp(av), axis=0, keepdims=True)
        ssum = sumexp - jnp.exp(tmax) + jnp.exp(tmin) + jnp.float32(1e-6)
        logp = jnp.log(jnp.exp(tmin) / ssum + jnp.float32(1e-6))
        o_con_ref[...] = (-jnp.sum(logp) * jnp.float32(1.0 / B)).reshape(1, 1)


@functools.partial(jax.jit, static_argnums=())
def _run(inputs, inputs2, idx_col, features, lab3):
    kern = pl.pallas_call(
        _body,
        grid=(NB,),
        in_specs=[
            pl.BlockSpec((TM, D), lambda i: (i, 0)),
            pl.BlockSpec((1, 1, TM), lambda i: (i, 0, 0),
                         memory_space=pltpu.SMEM),
            pl.BlockSpec((1, 1, TM), lambda i: (i, 0, 0)),
            pl.BlockSpec((B, D), lambda i: (0, 0)),
            pl.BlockSpec((B, D), lambda i: (0, 0)),
            pl.BlockSpec((B, 1), lambda i: (0, 0)),
        ],
        out_specs=[
            pl.BlockSpec((1, 1), lambda i: (0, 0)),
            pl.BlockSpec((1, 1), lambda i: (0, 0)),
        ],
        out_shape=[
            jax.ShapeDtypeStruct((1, 1), jnp.float32),
            jax.ShapeDtypeStruct((1, 1), jnp.float32),
        ],
        scratch_shapes=[
            pltpu.VMEM((C, B), jnp.float32),
            pltpu.VMEM((C, B), jnp.float32),
            pltpu.VMEM((C, D), jnp.float32),
            pltpu.VMEM((C, 1), jnp.float32),
            pltpu.VMEM((TM, B), jnp.float32),
            pltpu.VMEM((1, B), jnp.float32),
        ],
        compiler_params=pltpu.CompilerParams(
            dimension_semantics=("arbitrary",)),
    )
    o_con, o_mse = kern(features, lab3, lab3, inputs, inputs2, idx_col)
    return o_con, o_mse


def kernel(inputs, inputs2, indexes, source_classes, features, labels):
    lab3 = labels.astype(jnp.float32).reshape(NB, 1, TM)
    idx_col = indexes.astype(jnp.int32).reshape(B, 1)
    o_con, o_mse = _run(inputs, inputs2, idx_col, features, lab3)
    loss_con = o_con.reshape(())
    loss_mse = o_mse.reshape(()) / source_classes
    return loss_con, loss_mse


# speedup vs baseline: 2.7271x; 1.0657x over previous
import functools

import jax
import jax.numpy as jnp
from jax.experimental import pallas as pl
from jax.experimental.pallas import tpu as pltpu

TEMP = 0.05
B, D, M, C = 512, 128, 65536, 1000
TM = 1024
NB = M // TM
S = 500
NEG = -3e38
POS = 3e38


def _body(feat_ref, lab_s_ref, lab_v_ref, lab_c_ref, labf_ref, in1_ref,
          in2_ref, idx_ref, o_con_ref, o_mse_ref,
          amax, afsum, anum, lg_s, tgt, tmin_a, tmax_a):
    i = pl.program_id(0)

    @pl.when(i == 0)
    def _init():
        amax[...] = jnp.full_like(amax, NEG)
        afsum[...] = jnp.zeros_like(afsum)
        anum[...] = jnp.zeros_like(anum)
        tmin_a[...] = jnp.full_like(tmin_a, POS)
        tmax_a[...] = jnp.full_like(tmax_a, NEG)
        tgt[...] = jnp.zeros_like(tgt)

        def tstep(k, _):
            lane = jax.lax.broadcasted_iota(jnp.int32, (B, TM), 1) + k * TM
            ohk = (idx_ref[...] == lane).astype(jnp.float32)
            labk = labf_ref[:, pl.ds(k * TM, TM)]
            tgt[...] += jax.lax.dot_general(
                labk, ohk, dimension_numbers=(((1,), (1,)), ((), ())),
                preferred_element_type=jnp.float32)
            return 0
        jax.lax.fori_loop(0, NB, tstep, 0)

    lg_s[...] = jax.lax.dot_general(
        feat_ref[...], in1_ref[...],
        dimension_numbers=(((1,), (1,)), ((), ())),
        preferred_element_type=jnp.float32) * jnp.float32(1.0 / TEMP)
    lg = lg_s[...]

    mt = lab_c_ref[...] == tgt[...]
    tmin_a[...] = jnp.minimum(
        tmin_a[...], jnp.min(jnp.where(mt, lg, POS), axis=0, keepdims=True))
    tmax_a[...] = jnp.maximum(
        tmax_a[...], jnp.max(jnp.where(mt, lg, NEG), axis=0, keepdims=True))

    ciota = jax.lax.broadcasted_iota(jnp.int32, (1, C), 1).astype(jnp.float32)
    oh_mc = (lab_c_ref[...] == ciota).astype(jnp.float32)
    afsum[...] += jax.lax.dot_general(
        feat_ref[...], oh_mc, dimension_numbers=(((0,), (0,)), ((), ())),
        preferred_element_type=jnp.float32)
    anum[...] += jnp.sum(oh_mc, axis=0, keepdims=True)

    def mstep(m, _):
        c = lab_s_ref[0, 0, m].astype(jnp.int32)
        amax[pl.ds(c, 1), :] = jnp.maximum(amax[pl.ds(c, 1), :],
                                           lg_s[pl.ds(m, 1), :])
        return 0
    jax.lax.fori_loop(0, TM, mstep, 0)

    @pl.when(i == NB - 1)
    def _epilogue():
        inv_tn = jnp.float32(1.0 / TEMP) / anum[...]
        sim = jax.lax.dot_general(
            in1_ref[...], afsum[...],
            dimension_numbers=(((1,), (0,)), ((), ())),
            preferred_element_type=jnp.float32) * inv_tn
        sim2 = jax.lax.dot_general(
            in2_ref[...], afsum[...],
            dimension_numbers=(((1,), (0,)), ((), ())),
            preferred_element_type=jnp.float32) * inv_tn

        def rowsoftmax(p):
            mx = jnp.max(p, axis=1, keepdims=True)
            e = jnp.exp(p - mx)
            return e / jnp.sum(e, axis=1, keepdims=True)

        dsm = rowsoftmax(sim[:, :S]) - rowsoftmax(sim2[:, :S])
        o_mse_ref[...] = jnp.sum(dsm * dsm).reshape(1, 1)

        tmin = tmin_a[...]
        tmax = tmax_a[...]
        sumexp = jnp.sum(jnp.exp(amax[...]), axis=0, keepdims=True)
        ssum = sumexp - jnp.exp(tmax) + jnp.exp(tmin) + jnp.float32(1e-6)
        logp = jnp.log(jnp.exp(tmin) / ssum + jnp.float32(1e-6))
        o_con_ref[...] = (-jnp.sum(logp) * jnp.float32(1.0 / B)).reshape(1, 1)


@jax.jit
def _run(inputs, inputs2, idx_col, features, lab3, labc, labrow):
    kern = pl.pallas_call(
        _body,
        grid=(NB,),
        in_specs=[
            pl.BlockSpec((TM, D), lambda i: (i, 0)),
            pl.BlockSpec((1, 1, TM), lambda i: (i, 0, 0),
                         memory_space=pltpu.SMEM),
            pl.BlockSpec((1, 1, TM), lambda i: (i, 0, 0)),
            pl.BlockSpec((TM, 1), lambda i: (i, 0)),
            pl.BlockSpec((1, M), lambda i: (0, 0)),
            pl.BlockSpec((B, D), lambda i: (0, 0)),
            pl.BlockSpec((B, D), lambda i: (0, 0)),
            pl.BlockSpec((B, 1), lambda i: (0, 0)),
        ],
        out_specs=[
            pl.BlockSpec((1, 1), lambda i: (0, 0)),
            pl.BlockSpec((1, 1), lambda i: (0, 0)),
        ],
        out_shape=[
            jax.ShapeDtypeStruct((1, 1), jnp.float32),
            jax.ShapeDtypeStruct((1, 1), jnp.float32),
        ],
        scratch_shapes=[
            pltpu.VMEM((C, B), jnp.float32),
            pltpu.VMEM((D, C), jnp.float32),
            pltpu.VMEM((1, C), jnp.float32),
            pltpu.VMEM((TM, B), jnp.float32),
            pltpu.VMEM((1, B), jnp.float32),
            pltpu.VMEM((1, B), jnp.float32),
            pltpu.VMEM((1, B), jnp.float32),
        ],
        compiler_params=pltpu.CompilerParams(
            dimension_semantics=("arbitrary",)),
    )
    o_con, o_mse = kern(features, lab3, lab3, labc, labrow, inputs,
                        inputs2, idx_col)
    return o_con, o_mse


def kernel(inputs, inputs2, indexes, source_classes, features, labels):
    labf = labels.astype(jnp.float32)
    lab3 = labf.reshape(NB, 1, TM)
    labc = labf.reshape(M, 1)
    labrow = labf.reshape(1, M)
    idx_col = indexes.astype(jnp.int32).reshape(B, 1)
    o_con, o_mse = _run(inputs, inputs2, idx_col, features, lab3, labc, labrow)
    loss_con = o_con.reshape(())
    loss_mse = o_mse.reshape(()) / source_classes
    return loss_con, loss_mse


# speedup vs baseline: 4.6389x; 1.7011x over previous
import functools

import jax
import jax.numpy as jnp
from jax.experimental import pallas as pl
from jax.experimental.pallas import tpu as pltpu

TEMP = 0.05
B, D, M, C = 512, 128, 65536, 1000
TM = 1024
NB = M // TM
S = 500
NEG = -3e38
POS = 3e38
P = 8


def _body(feat_ref, lab_s_ref, lab_v_ref, lab_c_ref, labf_ref, in1_ref,
          in2_ref, idx_ref, o_con_ref, o_mse_ref,
          *scratch):
    amaxs = scratch[:P]
    afsum, anum, lg_s, tgt, tmin_a, tmax_a = scratch[P:]
    i = pl.program_id(0)

    @pl.when(i == 0)
    def _init():
        for p in range(P):
            amaxs[p][...] = jnp.full_like(amaxs[p], NEG)
        afsum[...] = jnp.zeros_like(afsum)
        anum[...] = jnp.zeros_like(anum)
        tmin_a[...] = jnp.full_like(tmin_a, POS)
        tmax_a[...] = jnp.full_like(tmax_a, NEG)
        tgt[...] = jnp.zeros_like(tgt)

        def tstep(k, _):
            lane = jax.lax.broadcasted_iota(jnp.int32, (B, TM), 1) + k * TM
            ohk = (idx_ref[...] == lane).astype(jnp.float32)
            labk = labf_ref[:, pl.ds(k * TM, TM)]
            tgt[...] += jax.lax.dot_general(
                labk, ohk, dimension_numbers=(((1,), (1,)), ((), ())),
                preferred_element_type=jnp.float32)
            return 0
        jax.lax.fori_loop(0, NB, tstep, 0)

    lg_s[...] = jax.lax.dot_general(
        feat_ref[...], in1_ref[...],
        dimension_numbers=(((1,), (1,)), ((), ())),
        preferred_element_type=jnp.float32) * jnp.float32(1.0 / TEMP)
    lg = lg_s[...]

    mt = lab_c_ref[...] == tgt[...]
    tmin_a[...] = jnp.minimum(
        tmin_a[...], jnp.min(jnp.where(mt, lg, POS), axis=0, keepdims=True))
    tmax_a[...] = jnp.maximum(
        tmax_a[...], jnp.max(jnp.where(mt, lg, NEG), axis=0, keepdims=True))

    ciota = jax.lax.broadcasted_iota(jnp.int32, (1, C), 1).astype(jnp.float32)
    oh_mc = (lab_c_ref[...] == ciota).astype(jnp.float32)
    afsum[...] += jax.lax.dot_general(
        feat_ref[...], oh_mc, dimension_numbers=(((0,), (0,)), ((), ())),
        preferred_element_type=jnp.float32)
    anum[...] += jnp.sum(oh_mc, axis=0, keepdims=True)

    def mstep(j, _):
        for p in range(P):
            m = j * P + p
            c = lab_s_ref[0, 0, m].astype(jnp.int32)
            amaxs[p][pl.ds(c, 1), :] = jnp.maximum(
                amaxs[p][pl.ds(c, 1), :], lg_s[pl.ds(m, 1), :])
        return 0
    jax.lax.fori_loop(0, TM // P, mstep, 0)

    @pl.when(i == NB - 1)
    def _epilogue():
        inv_tn = jnp.float32(1.0 / TEMP) / anum[...]
        sim = jax.lax.dot_general(
            in1_ref[...], afsum[...],
            dimension_numbers=(((1,), (0,)), ((), ())),
            preferred_element_type=jnp.float32) * inv_tn
        sim2 = jax.lax.dot_general(
            in2_ref[...], afsum[...],
            dimension_numbers=(((1,), (0,)), ((), ())),
            preferred_element_type=jnp.float32) * inv_tn

        def rowsoftmax(p):
            mx = jnp.max(p, axis=1, keepdims=True)
            e = jnp.exp(p - mx)
            return e / jnp.sum(e, axis=1, keepdims=True)

        dsm = rowsoftmax(sim[:, :S]) - rowsoftmax(sim2[:, :S])
        o_mse_ref[...] = jnp.sum(dsm * dsm).reshape(1, 1)

        tmin = tmin_a[...]
        tmax = tmax_a[...]
        amaxv = amaxs[0][...]
        for p in range(1, P):
            amaxv = jnp.maximum(amaxv, amaxs[p][...])
        sumexp = jnp.sum(jnp.exp(amaxv), axis=0, keepdims=True)
        ssum = sumexp - jnp.exp(tmax) + jnp.exp(tmin) + jnp.float32(1e-6)
        logp = jnp.log(jnp.exp(tmin) / ssum + jnp.float32(1e-6))
        o_con_ref[...] = (-jnp.sum(logp) * jnp.float32(1.0 / B)).reshape(1, 1)


@jax.jit
def _run(inputs, inputs2, idx_col, features, lab3, labc, labrow):
    kern = pl.pallas_call(
        _body,
        grid=(NB,),
        in_specs=[
            pl.BlockSpec((TM, D), lambda i: (i, 0)),
            pl.BlockSpec((1, 1, TM), lambda i: (i, 0, 0),
                         memory_space=pltpu.SMEM),
            pl.BlockSpec((1, 1, TM), lambda i: (i, 0, 0)),
            pl.BlockSpec((TM, 1), lambda i: (i, 0)),
            pl.BlockSpec((1, M), lambda i: (0, 0)),
            pl.BlockSpec((B, D), lambda i: (0, 0)),
            pl.BlockSpec((B, D), lambda i: (0, 0)),
            pl.BlockSpec((B, 1), lambda i: (0, 0)),
        ],
        out_specs=[
            pl.BlockSpec((1, 1), lambda i: (0, 0)),
            pl.BlockSpec((1, 1), lambda i: (0, 0)),
        ],
        out_shape=[
            jax.ShapeDtypeStruct((1, 1), jnp.float32),
            jax.ShapeDtypeStruct((1, 1), jnp.float32),
        ],
        scratch_shapes=[pltpu.VMEM((C, B), jnp.float32) for _ in range(P)] + [
            pltpu.VMEM((D, C), jnp.float32),
            pltpu.VMEM((1, C), jnp.float32),
            pltpu.VMEM((TM, B), jnp.float32),
            pltpu.VMEM((1, B), jnp.float32),
            pltpu.VMEM((1, B), jnp.float32),
            pltpu.VMEM((1, B), jnp.float32),
        ],
        compiler_params=pltpu.CompilerParams(
            dimension_semantics=("arbitrary",)),
    )
    o_con, o_mse = kern(features, lab3, lab3, labc, labrow, inputs,
                        inputs2, idx_col)
    return o_con, o_mse


def kernel(inputs, inputs2, indexes, source_classes, features, labels):
    labf = labels.astype(jnp.float32)
    lab3 = labf.reshape(NB, 1, TM)
    labc = labf.reshape(M, 1)
    labrow = labf.reshape(1, M)
    idx_col = indexes.astype(jnp.int32).reshape(B, 1)
    o_con, o_mse = _run(inputs, inputs2, idx_col, features, lab3, labc, labrow)
    loss_con = o_con.reshape(())
    loss_mse = o_mse.reshape(()) / source_classes
    return loss_con, loss_mse


# speedup vs baseline: 5.5605x; 1.1987x over previous
import functools

import jax
import jax.numpy as jnp
from jax.experimental import pallas as pl
from jax.experimental.pallas import tpu as pltpu

TEMP = 0.05
B, D, M, C = 512, 128, 65536, 1000
TM = 1024
NB = M // TM
S = 500
NEG = -3e38
POS = 3e38
P = 8


def _body(feat_ref, lab_s_ref, lab_v_ref, lab_c_ref, labf_ref, in1_ref,
          in2_ref, idx_ref, o_con_ref, o_mse_ref,
          *scratch):
    amaxs = scratch[:P]
    afsum, anum, lg_s, tgt, tmin_a, tmax_a = scratch[P:]
    i = pl.program_id(0)

    @pl.when(i == 0)
    def _init():
        for p in range(P):
            amaxs[p][...] = jnp.full_like(amaxs[p], NEG)
        afsum[...] = jnp.zeros_like(afsum)
        anum[...] = jnp.zeros_like(anum)
        tmin_a[...] = jnp.full_like(tmin_a, POS)
        tmax_a[...] = jnp.full_like(tmax_a, NEG)
        tgt[...] = jnp.zeros_like(tgt)

        def tstep(k, _):
            lane = jax.lax.broadcasted_iota(jnp.int32, (B, TM), 1) + k * TM
            ohk = (idx_ref[...] == lane).astype(jnp.float32)
            labk = labf_ref[:, pl.ds(k * TM, TM)]
            tgt[...] += jax.lax.dot_general(
                labk, ohk, dimension_numbers=(((1,), (1,)), ((), ())),
                preferred_element_type=jnp.float32)
            return 0
        jax.lax.fori_loop(0, NB, tstep, 0)

    lg_s[...] = jax.lax.dot_general(
        feat_ref[...], in1_ref[...],
        dimension_numbers=(((1,), (1,)), ((), ())),
        preferred_element_type=jnp.float32) * jnp.float32(1.0 / TEMP)
    lg = lg_s[...]

    mt = lab_c_ref[...] == tgt[...]
    tmin_a[...] = jnp.minimum(
        tmin_a[...], jnp.min(jnp.where(mt, lg, POS), axis=0, keepdims=True))
    tmax_a[...] = jnp.maximum(
        tmax_a[...], jnp.max(jnp.where(mt, lg, NEG), axis=0, keepdims=True))

    ciota = jax.lax.broadcasted_iota(jnp.int32, (1, C), 1).astype(jnp.float32)
    oh_mc = (lab_c_ref[...] == ciota).astype(jnp.float32)
    afsum[...] += jax.lax.dot_general(
        feat_ref[...], oh_mc, dimension_numbers=(((0,), (0,)), ((), ())),
        preferred_element_type=jnp.float32)
    anum[...] += jnp.sum(oh_mc, axis=0, keepdims=True)

    def mstep(j, _):
        blk = lg_s[pl.ds(j * P, P), :]
        for p in range(P):
            c = lab_s_ref[0, 0, j * P + p].astype(jnp.int32)
            row = jax.lax.slice_in_dim(blk, p, p + 1, axis=0)
            amaxs[p][pl.ds(c, 1), :] = jnp.maximum(
                amaxs[p][pl.ds(c, 1), :], row)
        return 0
    jax.lax.fori_loop(0, TM // P, mstep, 0)

    @pl.when(i == NB - 1)
    def _epilogue():
        inv_tn = jnp.float32(1.0 / TEMP) / anum[...]
        sim = jax.lax.dot_general(
            in1_ref[...], afsum[...],
            dimension_numbers=(((1,), (0,)), ((), ())),
            preferred_element_type=jnp.float32) * inv_tn
        sim2 = jax.lax.dot_general(
            in2_ref[...], afsum[...],
            dimension_numbers=(((1,), (0,)), ((), ())),
            preferred_element_type=jnp.float32) * inv_tn

        def rowsoftmax(p):
            mx = jnp.max(p, axis=1, keepdims=True)
            e = jnp.exp(p - mx)
            return e / jnp.sum(e, axis=1, keepdims=True)

        dsm = rowsoftmax(sim[:, :S]) - rowsoftmax(sim2[:, :S])
        o_mse_ref[...] = jnp.sum(dsm * dsm).reshape(1, 1)

        tmin = tmin_a[...]
        tmax = tmax_a[...]
        amaxv = amaxs[0][...]
        for p in range(1, P):
            amaxv = jnp.maximum(amaxv, amaxs[p][...])
        sumexp = jnp.sum(jnp.exp(amaxv), axis=0, keepdims=True)
        ssum = sumexp - jnp.exp(tmax) + jnp.exp(tmin) + jnp.float32(1e-6)
        logp = jnp.log(jnp.exp(tmin) / ssum + jnp.float32(1e-6))
        o_con_ref[...] = (-jnp.sum(logp) * jnp.float32(1.0 / B)).reshape(1, 1)


@jax.jit
def _run(inputs, inputs2, idx_col, features, lab3, labc, labrow):
    kern = pl.pallas_call(
        _body,
        grid=(NB,),
        in_specs=[
            pl.BlockSpec((TM, D), lambda i: (i, 0)),
            pl.BlockSpec((1, 1, TM), lambda i: (i, 0, 0),
                         memory_space=pltpu.SMEM),
            pl.BlockSpec((1, 1, TM), lambda i: (i, 0, 0)),
            pl.BlockSpec((TM, 1), lambda i: (i, 0)),
            pl.BlockSpec((1, M), lambda i: (0, 0)),
            pl.BlockSpec((B, D), lambda i: (0, 0)),
            pl.BlockSpec((B, D), lambda i: (0, 0)),
            pl.BlockSpec((B, 1), lambda i: (0, 0)),
        ],
        out_specs=[
            pl.BlockSpec((1, 1), lambda i: (0, 0)),
            pl.BlockSpec((1, 1), lambda i: (0, 0)),
        ],
        out_shape=[
            jax.ShapeDtypeStruct((1, 1), jnp.float32),
            jax.ShapeDtypeStruct((1, 1), jnp.float32),
        ],
        scratch_shapes=[pltpu.VMEM((C, B), jnp.float32) for _ in range(P)] + [
            pltpu.VMEM((D, C), jnp.float32),
            pltpu.VMEM((1, C), jnp.float32),
            pltpu.VMEM((TM, B), jnp.float32),
            pltpu.VMEM((1, B), jnp.float32),
            pltpu.VMEM((1, B), jnp.float32),
            pltpu.VMEM((1, B), jnp.float32),
        ],
        compiler_params=pltpu.CompilerParams(
            dimension_semantics=("arbitrary",)),
    )
    o_con, o_mse = kern(features, lab3, lab3, labc, labrow, inputs,
                        inputs2, idx_col)
    return o_con, o_mse


def kernel(inputs, inputs2, indexes, source_classes, features, labels):
    labf = labels.astype(jnp.float32)
    lab3 = labf.reshape(NB, 1, TM)
    labc = labf.reshape(M, 1)
    labrow = labf.reshape(1, M)
    idx_col = indexes.astype(jnp.int32).reshape(B, 1)
    o_con, o_mse = _run(inputs, inputs2, idx_col, features, lab3, labc, labrow)
    loss_con = o_con.reshape(())
    loss_mse = o_mse.reshape(()) / source_classes
    return loss_con, loss_mse


# speedup vs baseline: 5.6730x; 1.0202x over previous
import functools

import jax
import jax.numpy as jnp
from jax.experimental import pallas as pl
from jax.experimental.pallas import tpu as pltpu

TEMP = 0.05
B, D, M, C = 512, 128, 65536, 1000
TM = 1024
NB = M // TM
S = 500
NEG = -3e38
POS = 3e38
P = 16


def _body(feat_ref, lab_s_ref, lab_v_ref, lab_c_ref, labf_ref, in1_ref,
          in2_ref, idx_ref, o_con_ref, o_mse_ref,
          *scratch):
    amaxs = scratch[:P]
    afsum, anum, lg_s, tgt, tmin_a, tmax_a = scratch[P:]
    i = pl.program_id(0)

    @pl.when(i == 0)
    def _init():
        for p in range(P):
            amaxs[p][...] = jnp.full_like(amaxs[p], NEG)
        afsum[...] = jnp.zeros_like(afsum)
        anum[...] = jnp.zeros_like(anum)
        tmin_a[...] = jnp.full_like(tmin_a, POS)
        tmax_a[...] = jnp.full_like(tmax_a, NEG)
        tgt[...] = jnp.zeros_like(tgt)

        def tstep(k, _):
            lane = jax.lax.broadcasted_iota(jnp.int32, (B, TM), 1) + k * TM
            ohk = (idx_ref[...] == lane).astype(jnp.float32)
            labk = labf_ref[:, pl.ds(k * TM, TM)]
            tgt[...] += jax.lax.dot_general(
                labk, ohk, dimension_numbers=(((1,), (1,)), ((), ())),
                preferred_element_type=jnp.float32)
            return 0
        jax.lax.fori_loop(0, NB, tstep, 0)

    lg_s[...] = jax.lax.dot_general(
        feat_ref[...], in1_ref[...],
        dimension_numbers=(((1,), (1,)), ((), ())),
        preferred_element_type=jnp.float32) * jnp.float32(1.0 / TEMP)
    lg = lg_s[...]

    mt = lab_c_ref[...] == tgt[...]
    tmin_a[...] = jnp.minimum(
        tmin_a[...], jnp.min(jnp.where(mt, lg, POS), axis=0, keepdims=True))
    tmax_a[...] = jnp.maximum(
        tmax_a[...], jnp.max(jnp.where(mt, lg, NEG), axis=0, keepdims=True))

    ciota = jax.lax.broadcasted_iota(jnp.int32, (1, C), 1).astype(jnp.float32)
    oh_mc = (lab_c_ref[...] == ciota).astype(jnp.float32)
    afsum[...] += jax.lax.dot_general(
        feat_ref[...], oh_mc, dimension_numbers=(((0,), (0,)), ((), ())),
        preferred_element_type=jnp.float32)
    anum[...] += jnp.sum(oh_mc, axis=0, keepdims=True)

    def mstep(j, _):
        for g in range(P // 8):
            blk = lg_s[pl.ds(j * P + g * 8, 8), :]
            for q in range(8):
                p = g * 8 + q
                c = lab_s_ref[0, 0, j * P + p].astype(jnp.int32)
                row = jax.lax.slice_in_dim(blk, q, q + 1, axis=0)
                amaxs[p][pl.ds(c, 1), :] = jnp.maximum(
                    amaxs[p][pl.ds(c, 1), :], row)
        return 0
    jax.lax.fori_loop(0, TM // P, mstep, 0)

    @pl.when(i == NB - 1)
    def _epilogue():
        inv_tn = jnp.float32(1.0 / TEMP) / anum[...]
        sim = jax.lax.dot_general(
            in1_ref[...], afsum[...],
            dimension_numbers=(((1,), (0,)), ((), ())),
            preferred_element_type=jnp.float32) * inv_tn
        sim2 = jax.lax.dot_general(
            in2_ref[...], afsum[...],
            dimension_numbers=(((1,), (0,)), ((), ())),
            preferred_element_type=jnp.float32) * inv_tn

        def rowsoftmax(p):
            mx = jnp.max(p, axis=1, keepdims=True)
            e = jnp.exp(p - mx)
            return e / jnp.sum(e, axis=1, keepdims=True)

        dsm = rowsoftmax(sim[:, :S]) - rowsoftmax(sim2[:, :S])
        o_mse_ref[...] = jnp.sum(dsm * dsm).reshape(1, 1)

        tmin = tmin_a[...]
        tmax = tmax_a[...]
        amaxv = amaxs[0][...]
        for p in range(1, P):
            amaxv = jnp.maximum(amaxv, amaxs[p][...])
        sumexp = jnp.sum(jnp.exp(amaxv), axis=0, keepdims=True)
        ssum = sumexp - jnp.exp(tmax) + jnp.exp(tmin) + jnp.float32(1e-6)
        logp = jnp.log(jnp.exp(tmin) / ssum + jnp.float32(1e-6))
        o_con_ref[...] = (-jnp.sum(logp) * jnp.float32(1.0 / B)).reshape(1, 1)


@jax.jit
def _run(inputs, inputs2, idx_col, features, lab3, labc, labrow):
    kern = pl.pallas_call(
        _body,
        grid=(NB,),
        in_specs=[
            pl.BlockSpec((TM, D), lambda i: (i, 0)),
            pl.BlockSpec((1, 1, TM), lambda i: (i, 0, 0),
                         memory_space=pltpu.SMEM),
            pl.BlockSpec((1, 1, TM), lambda i: (i, 0, 0)),
            pl.BlockSpec((TM, 1), lambda i: (i, 0)),
            pl.BlockSpec((1, M), lambda i: (0, 0)),
            pl.BlockSpec((B, D), lambda i: (0, 0)),
            pl.BlockSpec((B, D), lambda i: (0, 0)),
            pl.BlockSpec((B, 1), lambda i: (0, 0)),
        ],
        out_specs=[
            pl.BlockSpec((1, 1), lambda i: (0, 0)),
            pl.BlockSpec((1, 1), lambda i: (0, 0)),
        ],
        out_shape=[
            jax.ShapeDtypeStruct((1, 1), jnp.float32),
            jax.ShapeDtypeStruct((1, 1), jnp.float32),
        ],
        scratch_shapes=[pltpu.VMEM((C, B), jnp.float32) for _ in range(P)] + [
            pltpu.VMEM((D, C), jnp.float32),
            pltpu.VMEM((1, C), jnp.float32),
            pltpu.VMEM((TM, B), jnp.float32),
            pltpu.VMEM((1, B), jnp.float32),
            pltpu.VMEM((1, B), jnp.float32),
            pltpu.VMEM((1, B), jnp.float32),
        ],
        compiler_params=pltpu.CompilerParams(
            dimension_semantics=("arbitrary",)),
    )
    o_con, o_mse = kern(features, lab3, lab3, labc, labrow, inputs,
                        inputs2, idx_col)
    return o_con, o_mse


def kernel(inputs, inputs2, indexes, source_classes, features, labels):
    labf = labels.astype(jnp.float32)
    lab3 = labf.reshape(NB, 1, TM)
    labc = labf.reshape(M, 1)
    labrow = labf.reshape(1, M)
    idx_col = indexes.astype(jnp.int32).reshape(B, 1)
    o_con, o_mse = _run(inputs, inputs2, idx_col, features, lab3, labc, labrow)
    loss_con = o_con.reshape(())
    loss_mse = o_mse.reshape(()) / source_classes
    return loss_con, loss_mse


# speedup vs baseline: 6.4916x; 1.1443x over previous
import functools

import jax
import jax.numpy as jnp
from jax.experimental import pallas as pl
from jax.experimental.pallas import tpu as pltpu

TEMP = 0.05
B, D, M, C = 512, 128, 65536, 1000
TM = 1024
NB = M // TM
S = 500
NEG = -3e38
POS = 3e38
P = 16


def _body(feat_ref, lab_s_ref, lab_v_ref, lab_c_ref, labf_ref, in1_ref,
          in2_ref, idx_ref, o_con_ref, o_mse_ref,
          *scratch):
    amaxs = scratch[:P]
    afsum, anum, lg_s, tgt, tmin_a, tmax_a = scratch[P:]
    i = pl.program_id(0)

    @pl.when(i == 0)
    def _init():
        for p in range(P):
            amaxs[p][...] = jnp.full_like(amaxs[p], NEG)
        afsum[...] = jnp.zeros_like(afsum)
        anum[...] = jnp.zeros_like(anum)
        tmin_a[...] = jnp.full_like(tmin_a, POS)
        tmax_a[...] = jnp.full_like(tmax_a, NEG)
        tgt[...] = jnp.zeros_like(tgt)

        def tstep(k, _):
            lane = jax.lax.broadcasted_iota(jnp.int32, (B, TM), 1) + k * TM
            ohk = (idx_ref[...] == lane).astype(jnp.float32)
            labk = labf_ref[:, pl.ds(k * TM, TM)]
            tgt[...] += jax.lax.dot_general(
                labk, ohk, dimension_numbers=(((1,), (1,)), ((), ())),
                preferred_element_type=jnp.float32)
            return 0
        jax.lax.fori_loop(0, NB, tstep, 0)

    lg_s[...] = jax.lax.dot_general(
        feat_ref[...], in1_ref[...],
        dimension_numbers=(((1,), (1,)), ((), ())),
        preferred_element_type=jnp.float32) * jnp.float32(1.0 / TEMP)
    lg = lg_s[...]

    mt = lab_c_ref[...] == tgt[...]
    tmin_a[...] = jnp.minimum(
        tmin_a[...], jnp.min(jnp.where(mt, lg, POS), axis=0, keepdims=True))
    tmax_a[...] = jnp.maximum(
        tmax_a[...], jnp.max(jnp.where(mt, lg, NEG), axis=0, keepdims=True))

    ciota = jax.lax.broadcasted_iota(jnp.int32, (1, C), 1).astype(jnp.float32)
    oh_mc = (lab_c_ref[...] == ciota).astype(jnp.float32)
    afsum[...] += jax.lax.dot_general(
        feat_ref[...], oh_mc, dimension_numbers=(((0,), (0,)), ((), ())),
        preferred_element_type=jnp.float32)
    anum[...] += jnp.sum(oh_mc, axis=0, keepdims=True)

    def mstep(j, _):
        for g in range(P // 8):
            blk = lg_s[pl.ds(j * P + g * 8, 8), :]
            blk3 = blk.reshape(8, 4, B // 4)
            for q in range(8):
                p = g * 8 + q
                c = lab_s_ref[0, 0, j * P + p].astype(jnp.int32)
                row = jax.lax.slice_in_dim(blk3, q, q + 1, axis=0)
                amaxs[p][pl.ds(c, 1)] = jnp.maximum(
                    amaxs[p][pl.ds(c, 1)], row)
        return 0
    jax.lax.fori_loop(0, TM // P, mstep, 0)

    @pl.when(i == NB - 1)
    def _epilogue():
        inv_tn = jnp.float32(1.0 / TEMP) / anum[...]
        sim = jax.lax.dot_general(
            in1_ref[...], afsum[...],
            dimension_numbers=(((1,), (0,)), ((), ())),
            preferred_element_type=jnp.float32) * inv_tn
        sim2 = jax.lax.dot_general(
            in2_ref[...], afsum[...],
            dimension_numbers=(((1,), (0,)), ((), ())),
            preferred_element_type=jnp.float32) * inv_tn

        def rowsoftmax(p):
            mx = jnp.max(p, axis=1, keepdims=True)
            e = jnp.exp(p - mx)
            return e / jnp.sum(e, axis=1, keepdims=True)

        dsm = rowsoftmax(sim[:, :S]) - rowsoftmax(sim2[:, :S])
        o_mse_ref[...] = jnp.sum(dsm * dsm).reshape(1, 1)

        tmin = tmin_a[...]
        tmax = tmax_a[...]
        amaxv = amaxs[0][...]
        for p in range(1, P):
            amaxv = jnp.maximum(amaxv, amaxs[p][...])
        s4 = jnp.sum(jnp.exp(amaxv), axis=0)
        sumexp = jnp.concatenate(
            [jax.lax.slice_in_dim(s4, r, r + 1, axis=0) for r in range(4)],
            axis=1)
        ssum = sumexp - jnp.exp(tmax) + jnp.exp(tmin) + jnp.float32(1e-6)
        logp = jnp.log(jnp.exp(tmin) / ssum + jnp.float32(1e-6))
        o_con_ref[...] = (-jnp.sum(logp) * jnp.float32(1.0 / B)).reshape(1, 1)


@jax.jit
def _run(inputs, inputs2, idx_col, features, lab3, labc, labrow):
    kern = pl.pallas_call(
        _body,
        grid=(NB,),
        in_specs=[
            pl.BlockSpec((TM, D), lambda i: (i, 0)),
            pl.BlockSpec((1, 1, TM), lambda i: (i, 0, 0),
                         memory_space=pltpu.SMEM),
            pl.BlockSpec((1, 1, TM), lambda i: (i, 0, 0)),
            pl.BlockSpec((TM, 1), lambda i: (i, 0)),
            pl.BlockSpec((1, M), lambda i: (0, 0)),
            pl.BlockSpec((B, D), lambda i: (0, 0)),
            pl.BlockSpec((B, D), lambda i: (0, 0)),
            pl.BlockSpec((B, 1), lambda i: (0, 0)),
        ],
        out_specs=[
            pl.BlockSpec((1, 1), lambda i: (0, 0)),
            pl.BlockSpec((1, 1), lambda i: (0, 0)),
        ],
        out_shape=[
            jax.ShapeDtypeStruct((1, 1), jnp.float32),
            jax.ShapeDtypeStruct((1, 1), jnp.float32),
        ],
        scratch_shapes=[pltpu.VMEM((C, 4, B // 4), jnp.float32)
                        for _ in range(P)] + [
            pltpu.VMEM((D, C), jnp.float32),
            pltpu.VMEM((1, C), jnp.float32),
            pltpu.VMEM((TM, B), jnp.float32),
            pltpu.VMEM((1, B), jnp.float32),
            pltpu.VMEM((1, B), jnp.float32),
            pltpu.VMEM((1, B), jnp.float32),
        ],
        compiler_params=pltpu.CompilerParams(
            dimension_semantics=("arbitrary",)),
    )
    o_con, o_mse = kern(features, lab3, lab3, labc, labrow, inputs,
                        inputs2, idx_col)
    return o_con, o_mse


def kernel(inputs, inputs2, indexes, source_classes, features, labels):
    labf = labels.astype(jnp.float32)
    lab3 = labf.reshape(NB, 1, TM)
    labc = labf.reshape(M, 1)
    labrow = labf.reshape(1, M)
    idx_col = indexes.astype(jnp.int32).reshape(B, 1)
    o_con, o_mse = _run(inputs, inputs2, idx_col, features, lab3, labc, labrow)
    loss_con = o_con.reshape(())
    loss_mse = o_mse.reshape(()) / source_classes
    return loss_con, loss_mse
